# baseline (speedup 1.0000x reference)
import jax
import jax.numpy as jnp
from jax import lax
from jax.experimental import pallas as pl
from jax.experimental.pallas import tpu as pltpu
from jax.experimental.pallas import tpu_sc as plsc

NUM_CLASSES = 1000
FEAT_DIM = 128
BATCH = 16384
CURVATURE = 1.0

_NC = 2
_NS = 16
_NW = _NC * _NS
_BPW = BATCH // _NW
_CH = 128
_NCH = _BPW // _CH
_L = 16
_RU = 4
_QR = FEAT_DIM // _L
_PCH = _CH * _L


def _sc_fused_body(feat_hbm, centers_hbm, label_hbm,
                   out_hbm,
                   idx_v, fbuf0, fbuf1, cbuf0, cbuf1,
                   dot_v, x2_v, y2_v,
                   semf0, semf1, semc0, semc1, semo):
    wid = lax.axis_index("s") * _NC + lax.axis_index("c")
    pltpu.sync_copy(label_hbm.at[wid], idx_v)

    fbufs = (fbuf0, fbuf1)
    cbufs = (cbuf0, cbuf1)
    semfs = (semf0, semf1)
    semcs = (semc0, semc1)

    def issue(j):
        p = j & 1
        cf = pltpu.async_copy(feat_hbm.at[wid, j], fbufs[p], semfs[p])
        cc = pltpu.async_copy(centers_hbm.at[idx_v.at[j]], cbufs[p], semcs[p])
        return (cf, cc)

    def writeback(j):
        sl = pl.ds(j * _PCH, _PCH)
        row = wid * _NCH + j
        return [pltpu.async_copy(dot_v.at[sl], out_hbm.at[row], semo),
                pltpu.async_copy(x2_v.at[sl], out_hbm.at[128 + row], semo),
                pltpu.async_copy(y2_v.at[sl], out_hbm.at[256 + row], semo)]

    pending = {0: issue(0)}
    outcopies = []
    for j in range(_NCH):
        if j + 1 < _NCH:
            pending[j + 1] = issue(j + 1)
        for c in pending.pop(j):
            c.wait()
        p = j & 1
        fb = fbufs[p]
        cb = cbufs[p]

        @plsc.parallel_loop(0, _CH, unroll=_RU)
        def _row_loop(r, fb=fb, cb=cb, j=j):
            dot0 = jnp.zeros((_L,), jnp.float32)
            dot1 = jnp.zeros((_L,), jnp.float32)
            x20 = jnp.zeros((_L,), jnp.float32)
            x21 = jnp.zeros((_L,), jnp.float32)
            y20 = jnp.zeros((_L,), jnp.float32)
            y21 = jnp.zeros((_L,), jnp.float32)
            for q in range(_QR):
                f = fb[r, pl.ds(q * _L, _L)]
                c = cb[r, pl.ds(q * _L, _L)]
                if q & 1:
                    dot1 = dot1 + f * c
                    x21 = x21 + f * f
                    y21 = y21 + c * c
                else:
                    dot0 = dot0 + f * c
                    x20 = x20 + f * f
                    y20 = y20 + c * c
            off = (j * _CH + r) * _L
            dot_v[pl.ds(off, _L)] = dot0 + dot1
            x2_v[pl.ds(off, _L)] = x20 + x21
            y2_v[pl.ds(off, _L)] = y20 + y21

        if j > 0:
            outcopies += writeback(j - 1)
    outcopies += writeback(_NCH - 1)
    for c in outcopies:
        c.wait()


def _sc_fused(feat4, centers, label3):
    mesh = plsc.VectorSubcoreMesh(core_axis_name="c", subcore_axis_name="s")
    out = jax.ShapeDtypeStruct((3 * _NW * _NCH, _PCH), jnp.float32)
    fn = pl.kernel(
        _sc_fused_body,
        out_type=out,
        mesh=mesh,
        scratch_types=[
            pltpu.VMEM((_NCH, _CH), jnp.int32),
            pltpu.VMEM((_CH, FEAT_DIM), jnp.float32),
            pltpu.VMEM((_CH, FEAT_DIM), jnp.float32),
            pltpu.VMEM((_CH, FEAT_DIM), jnp.float32),
            pltpu.VMEM((_CH, FEAT_DIM), jnp.float32),
            pltpu.VMEM((_BPW * _L,), jnp.float32),
            pltpu.VMEM((_BPW * _L,), jnp.float32),
            pltpu.VMEM((_BPW * _L,), jnp.float32),
            pltpu.SemaphoreType.DMA,
            pltpu.SemaphoreType.DMA,
            pltpu.SemaphoreType.DMA,
            pltpu.SemaphoreType.DMA,
            pltpu.SemaphoreType.DMA,
        ],
    )
    return fn(feat4, centers, label3)


_RK = BATCH * _L // 128


def _tc_finish_body(p_ref, out_ref):
    ii = lax.broadcasted_iota(jnp.int32, (_RK, 128), 0)
    jj = lax.broadcasted_iota(jnp.int32, (_RK, 128), 1)
    bsel = (ii // _L == jj).astype(jnp.float32)
    red = jnp.dot(p_ref[...], bsel, preferred_element_type=jnp.float32)
    dot = red[0:128, :]
    x2 = red[128:256, :]
    y2 = red[256:384, :]
    c = jnp.float32(CURVATURE)
    denom = 1.0 + c * x2 * y2 - 2.0 * c * dot + 1e-08
    a = 1.0 + c * y2
    b = 1.0 - c * x2
    num2 = a * a * x2 + b * b * y2 - 2.0 * a * b * dot
    num2 = jnp.maximum(num2, 0.0)
    sqrt_c = jnp.sqrt(c)
    norm = jnp.sqrt(num2) / denom
    z = jnp.clip(sqrt_c * norm, 1e-08, 1.0 - 1e-05)
    dist = (1.0 / sqrt_c) * jnp.log((1.0 + z) / (1.0 - z))
    out_ref[0, 0] = jnp.sum(dist) * jnp.float32(1.0 / BATCH)


def _tc_finish(p2):
    return pl.pallas_call(
        _tc_finish_body,
        out_specs=pl.BlockSpec(memory_space=pltpu.SMEM),
        out_shape=jax.ShapeDtypeStruct((1, 1), jnp.float32),
    )(p2)


def kernel(label, feat, centers):
    label3 = label.astype(jnp.int32).reshape(_NW, _NCH, _CH)
    feat4 = feat.reshape(_NW, _NCH, _CH, FEAT_DIM)
    parts = _sc_fused(feat4, centers, label3)
    loss = _tc_finish(parts)
    return loss[0, 0]


# speedup vs baseline: 1.0070x; 1.0070x over previous
import jax
import jax.numpy as jnp
from jax import lax
from jax.experimental import pallas as pl
from jax.experimental.pallas import tpu as pltpu
from jax.experimental.pallas import tpu_sc as plsc

NUM_CLASSES = 1000
FEAT_DIM = 128
BATCH = 16384
CURVATURE = 1.0

_NC = 2
_NS = 16
_NW = _NC * _NS
_BPW = BATCH // _NW
_CH = 128
_NCH = _BPW // _CH
_L = 16
_RU = 4
_QR = FEAT_DIM // _L
_PCH = _CH * _L


def _sc_fused_body(feat_hbm, centers_hbm, label_hbm,
                   out_hbm,
                   idx_v, fbuf0, fbuf1, cbuf0, cbuf1,
                   dot_v, x2_v, y2_v,
                   semf0, semf1, semc0, semc1, semo):
    wid = lax.axis_index("s") * _NC + lax.axis_index("c")

    fbufs = (fbuf0, fbuf1)
    cbufs = (cbuf0, cbuf1)
    semfs = (semf0, semf1)
    semcs = (semc0, semc1)

    def issue_feat(j):
        p = j & 1
        return pltpu.async_copy(feat_hbm.at[wid, j], fbufs[p], semfs[p])

    def issue_gather(j):
        p = j & 1
        return pltpu.async_copy(centers_hbm.at[idx_v.at[j]], cbufs[p],
                                semcs[p])

    lcopy = pltpu.async_copy(label_hbm.at[wid], idx_v, semo)
    fcopies = {0: issue_feat(0), 1: issue_feat(1)}
    lcopy.wait()
    gcopies = {0: issue_gather(0), 1: issue_gather(1)}


    def writeback(j):
        sl = pl.ds(j * _PCH, _PCH)
        row = wid * _NCH + j
        return [pltpu.async_copy(dot_v.at[sl], out_hbm.at[row], semo),
                pltpu.async_copy(x2_v.at[sl], out_hbm.at[128 + row], semo),
                pltpu.async_copy(y2_v.at[sl], out_hbm.at[256 + row], semo)]

    outcopies = []
    for j in range(_NCH):
        fcopies.pop(j).wait()
        gcopies.pop(j).wait()
        p = j & 1
        fb = fbufs[p]
        cb = cbufs[p]

        @plsc.parallel_loop(0, _CH, unroll=_RU)
        def _row_loop(r, fb=fb, cb=cb, j=j):
            dot0 = jnp.zeros((_L,), jnp.float32)
            dot1 = jnp.zeros((_L,), jnp.float32)
            x20 = jnp.zeros((_L,), jnp.float32)
            x21 = jnp.zeros((_L,), jnp.float32)
            y20 = jnp.zeros((_L,), jnp.float32)
            y21 = jnp.zeros((_L,), jnp.float32)
            for q in range(_QR):
                f = fb[r, pl.ds(q * _L, _L)]
                c = cb[r, pl.ds(q * _L, _L)]
                if q & 1:
                    dot1 = dot1 + f * c
                    x21 = x21 + f * f
                    y21 = y21 + c * c
                else:
                    dot0 = dot0 + f * c
                    x20 = x20 + f * f
                    y20 = y20 + c * c
            off = (j * _CH + r) * _L
            dot_v[pl.ds(off, _L)] = dot0 + dot1
            x2_v[pl.ds(off, _L)] = x20 + x21
            y2_v[pl.ds(off, _L)] = y20 + y21

        if j + 2 < _NCH:
            fcopies[j + 2] = issue_feat(j + 2)
            gcopies[j + 2] = issue_gather(j + 2)
        if j > 0:
            outcopies += writeback(j - 1)
    outcopies += writeback(_NCH - 1)
    for c in outcopies:
        c.wait()


def _sc_fused(feat4, centers, label3):
    mesh = plsc.VectorSubcoreMesh(core_axis_name="c", subcore_axis_name="s")
    out = jax.ShapeDtypeStruct((3 * _NW * _NCH, _PCH), jnp.float32)
    fn = pl.kernel(
        _sc_fused_body,
        out_type=out,
        mesh=mesh,
        scratch_types=[
            pltpu.VMEM((_NCH, _CH), jnp.int32),
            pltpu.VMEM((_CH, FEAT_DIM), jnp.float32),
            pltpu.VMEM((_CH, FEAT_DIM), jnp.float32),
            pltpu.VMEM((_CH, FEAT_DIM), jnp.float32),
            pltpu.VMEM((_CH, FEAT_DIM), jnp.float32),
            pltpu.VMEM((_BPW * _L,), jnp.float32),
            pltpu.VMEM((_BPW * _L,), jnp.float32),
            pltpu.VMEM((_BPW * _L,), jnp.float32),
            pltpu.SemaphoreType.DMA,
            pltpu.SemaphoreType.DMA,
            pltpu.SemaphoreType.DMA,
            pltpu.SemaphoreType.DMA,
            pltpu.SemaphoreType.DMA,
        ],
    )
    return fn(feat4, centers, label3)


_RK = BATCH * _L // 128


def _tc_finish_body(p_ref, out_ref):
    ii = lax.broadcasted_iota(jnp.int32, (_RK, 128), 0)
    jj = lax.broadcasted_iota(jnp.int32, (_RK, 128), 1)
    bsel = (ii // _L == jj).astype(jnp.float32)
    red = jnp.dot(p_ref[...], bsel, preferred_element_type=jnp.float32)
    dot = red[0:128, :]
    x2 = red[128:256, :]
    y2 = red[256:384, :]
    c = jnp.float32(CURVATURE)
    denom = 1.0 + c * x2 * y2 - 2.0 * c * dot + 1e-08
    a = 1.0 + c * y2
    b = 1.0 - c * x2
    num2 = a * a * x2 + b * b * y2 - 2.0 * a * b * dot
    num2 = jnp.maximum(num2, 0.0)
    sqrt_c = jnp.sqrt(c)
    norm = jnp.sqrt(num2) / denom
    z = jnp.clip(sqrt_c * norm, 1e-08, 1.0 - 1e-05)
    dist = (1.0 / sqrt_c) * jnp.log((1.0 + z) / (1.0 - z))
    out_ref[0, 0] = jnp.sum(dist) * jnp.float32(1.0 / BATCH)


def _tc_finish(p2):
    return pl.pallas_call(
        _tc_finish_body,
        out_specs=pl.BlockSpec(memory_space=pltpu.SMEM),
        out_shape=jax.ShapeDtypeStruct((1, 1), jnp.float32),
    )(p2)


def kernel(label, feat, centers):
    label3 = label.astype(jnp.int32).reshape(_NW, _NCH, _CH)
    feat4 = feat.reshape(_NW, _NCH, _CH, FEAT_DIM)
    parts = _sc_fused(feat4, centers, label3)
    loss = _tc_finish(parts)
    return loss[0, 0]
